# baseline (speedup 1.0000x reference)
import functools

import jax
import jax.numpy as jnp
from jax.experimental import pallas as pl
import jax.experimental.pallas.tpu as pltpu


def _router_kernel(x_ref, w_ref, b_ref, ssm_ref, msm_ref, out_ref, acc_ref,
                   *, grid, inv_n):
    i = pl.program_id(0)
    lt = jax.lax.dot_general(
        w_ref[...].astype(jnp.bfloat16), x_ref[...].astype(jnp.bfloat16),
        (((1,), (1,)), ((), ())),
        preferred_element_type=jnp.float32,
    ) + b_ref[...]
    m = jnp.max(lt, axis=0, keepdims=True)
    e = jnp.exp(lt - m)
    s = jnp.sum(e, axis=0, keepdims=True)
    p = e / s
    part = jnp.sum(p, axis=1, keepdims=True)

    @pl.when(i == 0)
    def _init():
        acc_ref[...] = part

    @pl.when(i > 0)
    def _acc():
        acc_ref[...] += part

    @pl.when(i == grid - 1)
    def _finish():
        nssm = ssm_ref.shape[0]
        avg = acc_ref[...] * inv_n
        out_ref[0:nssm, :] = ssm_ref[...] * avg[0:nssm, :]
        out_ref[nssm:, :] = msm_ref[...] * avg[nssm:, :]


def kernel(input_stream, ssm_slots, msm_slots, router_w, router_b,
           compress_w, compress_b):
    del compress_w, compress_b
    n, d = input_stream.shape
    nslots = router_w.shape[0]

    chunk = 4096
    grid = n // chunk

    out = pl.pallas_call(
        functools.partial(_router_kernel, grid=grid, inv_n=1.0 / n),
        grid=(grid,),
        in_specs=[
            pl.BlockSpec((chunk, d), lambda i: (i, 0)),
            pl.BlockSpec((nslots, d), lambda i: (0, 0)),
            pl.BlockSpec((nslots, 1), lambda i: (0, 0)),
            pl.BlockSpec(ssm_slots.shape, lambda i: (0, 0)),
            pl.BlockSpec(msm_slots.shape, lambda i: (0, 0)),
        ],
        out_specs=pl.BlockSpec((nslots, d), lambda i: (0, 0)),
        out_shape=jax.ShapeDtypeStruct((nslots, d), jnp.float32),
        scratch_shapes=[pltpu.VMEM((nslots, 1), jnp.float32)],
    )(input_stream, router_w, router_b.reshape(nslots, 1),
      ssm_slots, msm_slots)
    return out


# speedup vs baseline: 1.1367x; 1.1367x over previous
import functools

import jax
import jax.numpy as jnp
from jax.experimental import pallas as pl
import jax.experimental.pallas.tpu as pltpu


def _router_kernel(x_ref, w_ref, b_ref, ssm_ref, msm_ref, out_ref, acc_ref,
                   *, grid, inv_n):
    i = pl.program_id(0)
    lt = jax.lax.dot_general(
        w_ref[...].astype(jnp.bfloat16), x_ref[...].astype(jnp.bfloat16),
        (((1,), (1,)), ((), ())),
        preferred_element_type=jnp.float32,
    ) + b_ref[...]
    m = jnp.max(lt, axis=0, keepdims=True)
    e = jnp.exp(lt - m)
    s = jnp.sum(e, axis=0, keepdims=True)
    p = e / s
    part = jnp.sum(p, axis=1, keepdims=True)

    @pl.when(i == 0)
    def _init():
        acc_ref[...] = part

    @pl.when(i > 0)
    def _acc():
        acc_ref[...] += part

    @pl.when(i == grid - 1)
    def _finish():
        nssm = ssm_ref.shape[0]
        avg = acc_ref[...] * inv_n
        out_ref[0:nssm, :] = ssm_ref[...] * avg[0:nssm, :]
        out_ref[nssm:, :] = msm_ref[...] * avg[nssm:, :]


def kernel(input_stream, ssm_slots, msm_slots, router_w, router_b,
           compress_w, compress_b):
    del compress_w, compress_b
    n, d = input_stream.shape
    nslots = router_w.shape[0]

    chunk = 16384
    grid = n // chunk

    out = pl.pallas_call(
        functools.partial(_router_kernel, grid=grid, inv_n=1.0 / n),
        grid=(grid,),
        in_specs=[
            pl.BlockSpec((chunk, d), lambda i: (i, 0)),
            pl.BlockSpec((nslots, d), lambda i: (0, 0)),
            pl.BlockSpec((nslots, 1), lambda i: (0, 0)),
            pl.BlockSpec(ssm_slots.shape, lambda i: (0, 0)),
            pl.BlockSpec(msm_slots.shape, lambda i: (0, 0)),
        ],
        out_specs=pl.BlockSpec((nslots, d), lambda i: (0, 0)),
        out_shape=jax.ShapeDtypeStruct((nslots, d), jnp.float32),
        scratch_shapes=[pltpu.VMEM((nslots, 1), jnp.float32)],
    )(input_stream, router_w, router_b.reshape(nslots, 1),
      ssm_slots, msm_slots)
    return out


# speedup vs baseline: 1.1901x; 1.0470x over previous
import functools

import jax
import jax.numpy as jnp
from jax.experimental import pallas as pl
import jax.experimental.pallas.tpu as pltpu

NBUF = 4
CHUNK = 4096


def _router_kernel(x_hbm, w_ref, b_ref, ssm_ref, msm_ref, out_ref,
                   buf_ref, sem, *, grid, chunk, inv_n):
    def copy(idx, slot):
        return pltpu.make_async_copy(
            x_hbm.at[pl.ds(idx * chunk, chunk), :],
            buf_ref.at[slot],
            sem.at[slot],
        )

    for k in range(min(NBUF - 1, grid)):
        copy(k, k).start()

    w16 = w_ref[...].astype(jnp.bfloat16)
    b = b_ref[...]

    def step(i, acc):
        slot = jax.lax.rem(i, NBUF)
        nxt = i + NBUF - 1

        @pl.when(nxt < grid)
        def _prefetch():
            copy(nxt, jax.lax.rem(nxt, NBUF)).start()

        copy(i, slot).wait()
        lt = jax.lax.dot_general(
            w16, buf_ref[slot].astype(jnp.bfloat16),
            (((1,), (1,)), ((), ())),
            preferred_element_type=jnp.float32,
        ) + b
        m = jnp.max(lt, axis=0, keepdims=True)
        e = jnp.exp(lt - m)
        s = jnp.sum(e, axis=0, keepdims=True)
        p = e / s
        return acc + jnp.sum(p, axis=1, keepdims=True)

    acc = jax.lax.fori_loop(
        0, grid, step, jnp.zeros((w_ref.shape[0], 1), jnp.float32))
    avg = acc * inv_n
    nssm = ssm_ref.shape[0]
    out_ref[0:nssm, :] = ssm_ref[...] * avg[0:nssm, :]
    out_ref[nssm:, :] = msm_ref[...] * avg[nssm:, :]


def kernel(input_stream, ssm_slots, msm_slots, router_w, router_b,
           compress_w, compress_b):
    del compress_w, compress_b
    n, d = input_stream.shape
    nslots = router_w.shape[0]
    grid = n // CHUNK

    out = pl.pallas_call(
        functools.partial(_router_kernel, grid=grid, chunk=CHUNK, inv_n=1.0 / n),
        in_specs=[
            pl.BlockSpec(memory_space=pl.ANY),
            pl.BlockSpec((nslots, d), lambda: (0, 0)),
            pl.BlockSpec((nslots, 1), lambda: (0, 0)),
            pl.BlockSpec(ssm_slots.shape, lambda: (0, 0)),
            pl.BlockSpec(msm_slots.shape, lambda: (0, 0)),
        ],
        out_specs=pl.BlockSpec((nslots, d), lambda: (0, 0)),
        out_shape=jax.ShapeDtypeStruct((nslots, d), jnp.float32),
        scratch_shapes=[
            pltpu.VMEM((NBUF, CHUNK, d), jnp.float32),
            pltpu.SemaphoreType.DMA((NBUF,)),
        ],
    )(input_stream, router_w, router_b.reshape(nslots, 1),
      ssm_slots, msm_slots)
    return out
